# baseline (speedup 1.0000x reference)
import jax
import jax.numpy as jnp
from jax.experimental import pallas as pl
from jax.experimental.pallas import tpu as pltpu
from jax.experimental.pallas import tpu_sc as plsc

N_ATOMS = 4096
N_CGS = 512
KNN = 32
B_BATCH = 4
LANES = 16

BN1 = 512
BN3 = 128
BN4 = 512
SC_ROWS = N_ATOMS // 32


def _k1_softmax_stats(logits_ref, xyzc_ref, soft_ref, colsum_ref, gtun_ref,
                      idx_ref, d3_ref):
    i = pl.program_id(0)
    x = logits_ref[...]
    m = jnp.max(x, axis=1, keepdims=True)
    e = jnp.exp(x - m)
    s = jnp.sum(e, axis=1, keepdims=True)
    soft = e / s
    soft_ref[...] = soft

    col = jax.lax.broadcasted_iota(jnp.int32, (BN1, N_CGS), 1)
    hit = jnp.where(x == m, col, N_CGS)
    idx_ref[...] = jnp.min(hit, axis=1, keepdims=True)

    softT = jnp.transpose(soft)
    part_cs = jnp.sum(softT, axis=1, keepdims=True)
    part_gt = jnp.dot(softT, xyzc_ref[...],
                      preferred_element_type=jnp.float32)

    @pl.when(i == 0)
    def _():
        colsum_ref[...] = part_cs
        gtun_ref[...] = part_gt

    @pl.when(i != 0)
    def _():
        colsum_ref[...] += part_cs
        gtun_ref[...] += part_gt

    @pl.when(i == pl.num_programs(0) - 1)
    def _():
        r = 1.0 / (colsum_ref[...] + 1e-8)
        gt = gtun_ref[...] * r
        g1 = jax.lax.slice(gt, (1, 0), (KNN + 1, LANES))
        d3_ref[...] = g1[None, :, :] - gt[:, None, :]


def _sc_broadcast(soft):
    mesh = plsc.VectorSubcoreMesh(core_axis_name="c", subcore_axis_name="s")

    @pl.kernel(
        out_type=jax.ShapeDtypeStruct((B_BATCH, N_ATOMS, N_CGS),
                                      jnp.float32),
        mesh=mesh,
        scratch_types=[pltpu.VMEM((SC_ROWS, N_CGS), jnp.float32)])
    def sc_kernel(soft_hbm, out_hbm, buf_ref):
        c = jax.lax.axis_index("c")
        s = jax.lax.axis_index("s")
        base = (c * 16 + s) * SC_ROWS
        rows = pl.ds(base, SC_ROWS)
        pltpu.sync_copy(soft_hbm.at[rows, :], buf_ref)
        for b in range(B_BATCH):
            pltpu.sync_copy(buf_ref, out_hbm.at[b, rows, :])

    return sc_kernel(soft)


def _k2_big_matmul(b_ref, d_ref, soft_ref, gtun_ref, colsum_ref,
                   dx_ref, vt_ref, tbl_ref):
    i = pl.program_id(0)
    dx = jnp.dot(b_ref[...], d_ref[...],
                 preferred_element_type=jnp.float32)
    dx_ref[...] = dx
    softT = jnp.transpose(soft_ref[...])
    part = jnp.dot(softT, dx, preferred_element_type=jnp.float32)

    @pl.when(i == 0)
    def _():
        vt_ref[...] = part

    @pl.when(i != 0)
    def _():
        vt_ref[...] += part

    @pl.when(i == pl.num_programs(0) - 1)
    def _():
        r = 1.0 / (colsum_ref[...] + 1e-8)
        tbl_ref[...] = (gtun_ref[...] - vt_ref[...]) * r


def _k3_gather_combine(idx_ref, tbl_ref, dx_ref, out_ref):
    col = jax.lax.broadcasted_iota(jnp.int32, (BN4, N_CGS), 1)
    onehot = (idx_ref[...] == col).astype(jnp.float32)
    out_ref[...] = jnp.dot(onehot, tbl_ref[...],
                           preferred_element_type=jnp.float32) + dx_ref[...]


def kernel(xyz, z, nbr_list, bonds, assign_logits, B_param):
    f32 = jnp.float32

    xyzc = jnp.pad(jnp.transpose(xyz, (1, 0, 2)),
                   ((0, 0), (0, 0), (0, 1))).reshape(N_ATOMS, LANES)

    grid1 = N_ATOMS // BN1
    soft, colsum, gt_un, idx_col, d3 = pl.pallas_call(
        _k1_softmax_stats,
        grid=(grid1,),
        in_specs=[
            pl.BlockSpec((BN1, N_CGS), lambda i: (i, 0)),
            pl.BlockSpec((BN1, LANES), lambda i: (i, 0)),
        ],
        out_specs=[
            pl.BlockSpec((BN1, N_CGS), lambda i: (i, 0)),
            pl.BlockSpec((N_CGS, 1), lambda i: (0, 0)),
            pl.BlockSpec((N_CGS, LANES), lambda i: (0, 0)),
            pl.BlockSpec((BN1, 1), lambda i: (i, 0)),
            pl.BlockSpec((N_CGS, KNN, LANES), lambda i: (0, 0, 0)),
        ],
        out_shape=[
            jax.ShapeDtypeStruct((N_ATOMS, N_CGS), f32),
            jax.ShapeDtypeStruct((N_CGS, 1), f32),
            jax.ShapeDtypeStruct((N_CGS, LANES), f32),
            jax.ShapeDtypeStruct((N_ATOMS, 1), jnp.int32),
            jax.ShapeDtypeStruct((N_CGS, KNN, LANES), f32),
        ],
    )(assign_logits, xyzc)

    soft_bcast = _sc_broadcast(soft)

    d_flat = d3.reshape(N_CGS * KNN, LANES)

    grid3 = N_ATOMS // BN3
    dx_all, vt, tbl = pl.pallas_call(
        _k2_big_matmul,
        grid=(grid3,),
        in_specs=[
            pl.BlockSpec((BN3, N_CGS * KNN), lambda i: (i, 0)),
            pl.BlockSpec((N_CGS * KNN, LANES), lambda i: (0, 0)),
            pl.BlockSpec((BN3, N_CGS), lambda i: (i, 0)),
            pl.BlockSpec((N_CGS, LANES), lambda i: (0, 0)),
            pl.BlockSpec((N_CGS, 1), lambda i: (0, 0)),
        ],
        out_specs=[
            pl.BlockSpec((BN3, LANES), lambda i: (i, 0)),
            pl.BlockSpec((N_CGS, LANES), lambda i: (0, 0)),
            pl.BlockSpec((N_CGS, LANES), lambda i: (0, 0)),
        ],
        out_shape=[
            jax.ShapeDtypeStruct((N_ATOMS, LANES), f32),
            jax.ShapeDtypeStruct((N_CGS, LANES), f32),
            jax.ShapeDtypeStruct((N_CGS, LANES), f32),
        ],
    )(B_param, d_flat, soft, gt_un, colsum)

    grid4 = N_ATOMS // BN4
    recon16 = pl.pallas_call(
        _k3_gather_combine,
        grid=(grid4,),
        in_specs=[
            pl.BlockSpec((BN4, 1), lambda i: (i, 0)),
            pl.BlockSpec((N_CGS, LANES), lambda i: (0, 0)),
            pl.BlockSpec((BN4, LANES), lambda i: (i, 0)),
        ],
        out_specs=pl.BlockSpec((BN4, LANES), lambda i: (i, 0)),
        out_shape=jax.ShapeDtypeStruct((N_ATOMS, LANES), f32),
    )(idx_col, tbl, dx_all)

    xyz_recon = jnp.transpose(
        recon16.reshape(N_ATOMS, B_BATCH, 4), (1, 0, 2))[:, :, :3]
    return (soft_bcast, xyz, xyz_recon)


# speedup vs baseline: 1.0052x; 1.0052x over previous
import dataclasses

import jax
import jax.numpy as jnp
from jax.experimental import pallas as pl
from jax.experimental.pallas import tpu as pltpu
from jax.experimental.pallas import tpu_sc as plsc

N_ATOMS = 4096
N_CGS = 512
KNN = 32
B_BATCH = 4
LANES = 16

BN1 = 512
BN3 = 128
BN4 = 512
SC_ROWS = N_ATOMS // 32


def _k1_softmax_stats(logits_ref, xyzc_ref, soft_ref, colsum_ref, gtun_ref,
                      idx_ref, d3_ref):
    i = pl.program_id(0)
    x = logits_ref[...]
    m = jnp.max(x, axis=1, keepdims=True)
    e = jnp.exp(x - m)
    s = jnp.sum(e, axis=1, keepdims=True)
    soft = e / s
    soft_ref[...] = soft

    col = jax.lax.broadcasted_iota(jnp.int32, (BN1, N_CGS), 1)
    hit = jnp.where(x == m, col, N_CGS)
    idx_ref[...] = jnp.min(hit, axis=1, keepdims=True)

    softT = jnp.transpose(soft)
    part_cs = jnp.sum(softT, axis=1, keepdims=True)
    part_gt = jnp.dot(softT, xyzc_ref[...],
                      preferred_element_type=jnp.float32)

    @pl.when(i == 0)
    def _():
        colsum_ref[...] = part_cs
        gtun_ref[...] = part_gt

    @pl.when(i != 0)
    def _():
        colsum_ref[...] += part_cs
        gtun_ref[...] += part_gt

    @pl.when(i == pl.num_programs(0) - 1)
    def _():
        r = 1.0 / (colsum_ref[...] + 1e-8)
        gt = gtun_ref[...] * r
        g1 = jax.lax.slice(gt, (1, 0), (KNN + 1, LANES))
        d3_ref[...] = g1[None, :, :] - gt[:, None, :]


def _sc_softmax_broadcast(logits):
    mesh = plsc.VectorSubcoreMesh(core_axis_name="c", subcore_axis_name="s")
    nch = N_CGS // 16
    cp = pltpu.CompilerParams()
    if "needs_layout_passes" in pltpu.CompilerParams.__dataclass_fields__:
        cp = dataclasses.replace(cp, needs_layout_passes=False)

    @pl.kernel(
        out_type=jax.ShapeDtypeStruct((B_BATCH, N_ATOMS, N_CGS),
                                      jnp.float32),
        mesh=mesh,
        compiler_params=cp,
        scratch_types=[pltpu.VMEM((SC_ROWS, N_CGS), jnp.float32)])
    def sc_kernel(logits_hbm, out_hbm, buf_ref):
        c = jax.lax.axis_index("c")
        s = jax.lax.axis_index("s")
        base = (c * 16 + s) * SC_ROWS
        rows = pl.ds(base, SC_ROWS)
        pltpu.sync_copy(logits_hbm.at[rows, :], buf_ref)

        @pl.loop(0, SC_ROWS)
        def _(r):
            def _chunk(k):
                return buf_ref.at[r, pl.ds(k * 16, 16)]

            mv = jax.lax.fori_loop(
                1, nch, lambda k, m: jnp.maximum(m, _chunk(k)[...]),
                _chunk(0)[...])
            m = jnp.max(mv)

            def _expsum(k, sv):
                e = jnp.exp(_chunk(k)[...] - m)
                _chunk(k)[...] = e
                return sv + e

            sv = jax.lax.fori_loop(0, nch, _expsum,
                                   jnp.zeros((16,), jnp.float32))
            sve = jnp.zeros((16,), jnp.float32) + jnp.sum(sv)
            rinv = jnp.full((16,), 1.0, jnp.float32) / sve

            def _scale(k, carry):
                _chunk(k)[...] = _chunk(k)[...] * rinv
                return carry

            jax.lax.fori_loop(0, nch, _scale, 0)

        for b in range(B_BATCH):
            pltpu.sync_copy(buf_ref, out_hbm.at[b, rows, :])

    return sc_kernel(logits)


def _k2_big_matmul(b_ref, d_ref, soft_ref, gtun_ref, colsum_ref,
                   dx_ref, vt_ref, tbl_ref):
    i = pl.program_id(0)
    dx = jnp.dot(b_ref[...], d_ref[...],
                 preferred_element_type=jnp.float32)
    dx_ref[...] = dx
    softT = jnp.transpose(soft_ref[...])
    part = jnp.dot(softT, dx, preferred_element_type=jnp.float32)

    @pl.when(i == 0)
    def _():
        vt_ref[...] = part

    @pl.when(i != 0)
    def _():
        vt_ref[...] += part

    @pl.when(i == pl.num_programs(0) - 1)
    def _():
        r = 1.0 / (colsum_ref[...] + 1e-8)
        tbl_ref[...] = (gtun_ref[...] - vt_ref[...]) * r


def _k3_gather_combine(idx_ref, tbl_ref, dx_ref, out_ref):
    col = jax.lax.broadcasted_iota(jnp.int32, (BN4, N_CGS), 1)
    onehot = (idx_ref[...] == col).astype(jnp.float32)
    out_ref[...] = jnp.dot(onehot, tbl_ref[...],
                           preferred_element_type=jnp.float32) + dx_ref[...]


def kernel(xyz, z, nbr_list, bonds, assign_logits, B_param):
    f32 = jnp.float32

    xyzc = jnp.pad(jnp.transpose(xyz, (1, 0, 2)),
                   ((0, 0), (0, 0), (0, 1))).reshape(N_ATOMS, LANES)

    grid1 = N_ATOMS // BN1
    soft, colsum, gt_un, idx_col, d3 = pl.pallas_call(
        _k1_softmax_stats,
        grid=(grid1,),
        in_specs=[
            pl.BlockSpec((BN1, N_CGS), lambda i: (i, 0)),
            pl.BlockSpec((BN1, LANES), lambda i: (i, 0)),
        ],
        out_specs=[
            pl.BlockSpec((BN1, N_CGS), lambda i: (i, 0)),
            pl.BlockSpec((N_CGS, 1), lambda i: (0, 0)),
            pl.BlockSpec((N_CGS, LANES), lambda i: (0, 0)),
            pl.BlockSpec((BN1, 1), lambda i: (i, 0)),
            pl.BlockSpec((N_CGS, KNN, LANES), lambda i: (0, 0, 0)),
        ],
        out_shape=[
            jax.ShapeDtypeStruct((N_ATOMS, N_CGS), f32),
            jax.ShapeDtypeStruct((N_CGS, 1), f32),
            jax.ShapeDtypeStruct((N_CGS, LANES), f32),
            jax.ShapeDtypeStruct((N_ATOMS, 1), jnp.int32),
            jax.ShapeDtypeStruct((N_CGS, KNN, LANES), f32),
        ],
    )(assign_logits, xyzc)

    soft_bcast = _sc_softmax_broadcast(assign_logits)

    d_flat = d3.reshape(N_CGS * KNN, LANES)

    grid3 = N_ATOMS // BN3
    dx_all, vt, tbl = pl.pallas_call(
        _k2_big_matmul,
        grid=(grid3,),
        in_specs=[
            pl.BlockSpec((BN3, N_CGS * KNN), lambda i: (i, 0)),
            pl.BlockSpec((N_CGS * KNN, LANES), lambda i: (0, 0)),
            pl.BlockSpec((BN3, N_CGS), lambda i: (i, 0)),
            pl.BlockSpec((N_CGS, LANES), lambda i: (0, 0)),
            pl.BlockSpec((N_CGS, 1), lambda i: (0, 0)),
        ],
        out_specs=[
            pl.BlockSpec((BN3, LANES), lambda i: (i, 0)),
            pl.BlockSpec((N_CGS, LANES), lambda i: (0, 0)),
            pl.BlockSpec((N_CGS, LANES), lambda i: (0, 0)),
        ],
        out_shape=[
            jax.ShapeDtypeStruct((N_ATOMS, LANES), f32),
            jax.ShapeDtypeStruct((N_CGS, LANES), f32),
            jax.ShapeDtypeStruct((N_CGS, LANES), f32),
        ],
    )(B_param, d_flat, soft, gt_un, colsum)

    grid4 = N_ATOMS // BN4
    recon16 = pl.pallas_call(
        _k3_gather_combine,
        grid=(grid4,),
        in_specs=[
            pl.BlockSpec((BN4, 1), lambda i: (i, 0)),
            pl.BlockSpec((N_CGS, LANES), lambda i: (0, 0)),
            pl.BlockSpec((BN4, LANES), lambda i: (i, 0)),
        ],
        out_specs=pl.BlockSpec((BN4, LANES), lambda i: (i, 0)),
        out_shape=jax.ShapeDtypeStruct((N_ATOMS, LANES), f32),
    )(idx_col, tbl, dx_all)

    xyz_recon = jnp.transpose(
        recon16.reshape(N_ATOMS, B_BATCH, 4), (1, 0, 2))[:, :, :3]
    return (soft_bcast, xyz, xyz_recon)
